# Initial kernel scaffold; baseline (speedup 1.0000x reference)
import jax
import jax.numpy as jnp
from jax.experimental import pallas as pl


def kernel(theta, enc_W1, enc_b1, enc_W2, enc_b2, edge_W1, edge_b1, edge_W2, edge_b2, node_W1, node_b1, node_W2, node_b2, glob_W1, glob_b1, glob_W2, glob_b2):
    raise NotImplementedError("write your pallas kernel here")


# speedup vs baseline: 59.9323x; 59.9323x over previous
import functools

import jax
import jax.numpy as jnp
from jax.experimental import pallas as pl
from jax.experimental.pallas import tpu as pltpu

B = 256
K = 32
INPUT_DIM = 128
LATENT_DIM = 256
NODE_DIM = 128
EDGE_DIM = 128
N_ACTIONS = 32
N = B * K

G_CHUNK = 32
ROWS = G_CHUNK * K


def _gn_kernel(theta_ref,
               enc_W1_ref, enc_b1_ref, enc_W2_ref, enc_b2_ref,
               eW1r_ref, eW1s_ref, edge_b1_ref, edge_W2_ref, edge_b2_ref,
               nW1e_ref, nW1h_ref, node_b1_ref, node_W2_ref, node_b2_ref,
               gW1e_ref, gW1n_ref, glob_b1_ref, glob_W2_ref, glob_b2_ref,
               out_ref):
    f32 = jnp.float32

    h1 = jnp.maximum(
        jnp.dot(theta_ref[...], enc_W1_ref[...], preferred_element_type=f32)
        + enc_b1_ref[...], 0.0)
    h = jnp.dot(h1, enc_W2_ref[...], preferred_element_type=f32) + enc_b2_ref[...]

    a = jnp.dot(h, eW1r_ref[...], preferred_element_type=f32) + edge_b1_ref[...]
    b = jnp.dot(h, eW1s_ref[...], preferred_element_type=f32)

    a3 = a.reshape(G_CHUNK, K, LATENT_DIM)
    b3 = b.reshape(G_CHUNK, K, LATENT_DIM)

    acc = -jnp.maximum(a3 + b3, 0.0)
    for s in range(K):
        acc = acc + jnp.maximum(a3 + b3[:, s:s + 1, :], 0.0)
    R = acc.reshape(ROWS, LATENT_DIM)

    recv_mean = (jnp.dot(R, edge_W2_ref[...], preferred_element_type=f32)
                 / float(K - 1)) + edge_b2_ref[...]

    v1 = jnp.maximum(
        jnp.dot(recv_mean, nW1e_ref[...], preferred_element_type=f32)
        + jnp.dot(h, nW1h_ref[...], preferred_element_type=f32)
        + node_b1_ref[...], 0.0)
    v = jnp.dot(v1, node_W2_ref[...], preferred_element_type=f32) + node_b2_ref[...]

    Rsum = jnp.sum(R.reshape(G_CHUNK, K, LATENT_DIM), axis=1)
    edge_agg = (jnp.dot(Rsum, edge_W2_ref[...], preferred_element_type=f32)
                / float(K * (K - 1))) + edge_b2_ref[...]
    node_agg = jnp.sum(v.reshape(G_CHUNK, K, NODE_DIM), axis=1) / float(K)

    g1 = jnp.maximum(
        jnp.dot(edge_agg, gW1e_ref[...], preferred_element_type=f32)
        + jnp.dot(node_agg, gW1n_ref[...], preferred_element_type=f32)
        + glob_b1_ref[...], 0.0)
    out_ref[...] = (jnp.dot(g1, glob_W2_ref[...], preferred_element_type=f32)
                    + glob_b2_ref[...])


@jax.jit
def kernel(theta, enc_W1, enc_b1, enc_W2, enc_b2, edge_W1, edge_b1, edge_W2,
           edge_b2, node_W1, node_b1, node_W2, node_b2, glob_W1, glob_b1,
           glob_W2, glob_b2):
    n_chunks = B // G_CHUNK

    def row2d(bias):
        return bias.reshape(1, -1)

    rep = lambda shape: pl.BlockSpec(shape, lambda i: (0, 0))

    grid_spec = pl.GridSpec(
        grid=(n_chunks,),
        in_specs=[
            pl.BlockSpec((ROWS, INPUT_DIM), lambda i: (i, 0)),
            rep((INPUT_DIM, LATENT_DIM)), rep((1, LATENT_DIM)),
            rep((LATENT_DIM, NODE_DIM)), rep((1, NODE_DIM)),
            rep((NODE_DIM, LATENT_DIM)), rep((NODE_DIM, LATENT_DIM)),
            rep((1, LATENT_DIM)),
            rep((LATENT_DIM, EDGE_DIM)), rep((1, EDGE_DIM)),
            rep((EDGE_DIM, LATENT_DIM)), rep((NODE_DIM, LATENT_DIM)),
            rep((1, LATENT_DIM)),
            rep((LATENT_DIM, NODE_DIM)), rep((1, NODE_DIM)),
            rep((EDGE_DIM, LATENT_DIM)), rep((NODE_DIM, LATENT_DIM)),
            rep((1, LATENT_DIM)),
            rep((LATENT_DIM, N_ACTIONS)), rep((1, N_ACTIONS)),
        ],
        out_specs=pl.BlockSpec((G_CHUNK, N_ACTIONS), lambda i: (i, 0)),
    )

    return pl.pallas_call(
        _gn_kernel,
        grid_spec=grid_spec,
        out_shape=jax.ShapeDtypeStruct((B, N_ACTIONS), jnp.float32),
        compiler_params=pltpu.CompilerParams(
            dimension_semantics=("arbitrary",)),
    )(
        theta,
        enc_W1, row2d(enc_b1), enc_W2, row2d(enc_b2),
        edge_W1[:NODE_DIM], edge_W1[NODE_DIM:], row2d(edge_b1),
        edge_W2, row2d(edge_b2),
        node_W1[:EDGE_DIM], node_W1[EDGE_DIM:], row2d(node_b1),
        node_W2, row2d(node_b2),
        glob_W1[:EDGE_DIM], glob_W1[EDGE_DIM:], row2d(glob_b1),
        glob_W2, row2d(glob_b2),
    )


# speedup vs baseline: 63.5340x; 1.0601x over previous
import functools

import jax
import jax.numpy as jnp
from jax.experimental import pallas as pl
from jax.experimental.pallas import tpu as pltpu

B = 256
K = 32
INPUT_DIM = 128
LATENT_DIM = 256
NODE_DIM = 128
EDGE_DIM = 128
N_ACTIONS = 32
N = B * K

G_CHUNK = 64
ROWS = G_CHUNK * K


def _gn_kernel(theta_ref,
               enc_W1_ref, enc_b1_ref, enc_W2_ref, enc_b2_ref,
               eW1r_ref, eW1s_ref, edge_b1_ref, edge_W2_ref, edge_b2_ref,
               nW1e_ref, nW1h_ref, node_b1_ref, node_W2_ref, node_b2_ref,
               gW1e_ref, gW1n_ref, glob_b1_ref, glob_W2_ref, glob_b2_ref,
               out_ref):
    f32 = jnp.float32

    h1 = jnp.maximum(
        jnp.dot(theta_ref[...], enc_W1_ref[...], preferred_element_type=f32)
        + enc_b1_ref[...], 0.0)
    h = jnp.dot(h1, enc_W2_ref[...], preferred_element_type=f32) + enc_b2_ref[...]

    a = jnp.dot(h, eW1r_ref[...], preferred_element_type=f32) + edge_b1_ref[...]
    b = jnp.dot(h, eW1s_ref[...], preferred_element_type=f32)

    a3 = a.reshape(G_CHUNK, K, LATENT_DIM)
    b3 = b.reshape(G_CHUNK, K, LATENT_DIM)

    acc = -jnp.maximum(a3 + b3, 0.0)
    for s in range(K):
        acc = acc + jnp.maximum(a3 + b3[:, s:s + 1, :], 0.0)
    R = acc.reshape(ROWS, LATENT_DIM)

    recv_mean = (jnp.dot(R, edge_W2_ref[...], preferred_element_type=f32)
                 / float(K - 1)) + edge_b2_ref[...]

    v1 = jnp.maximum(
        jnp.dot(recv_mean, nW1e_ref[...], preferred_element_type=f32)
        + jnp.dot(h, nW1h_ref[...], preferred_element_type=f32)
        + node_b1_ref[...], 0.0)
    v = jnp.dot(v1, node_W2_ref[...], preferred_element_type=f32) + node_b2_ref[...]

    Rsum = jnp.sum(R.reshape(G_CHUNK, K, LATENT_DIM), axis=1)
    edge_agg = (jnp.dot(Rsum, edge_W2_ref[...], preferred_element_type=f32)
                / float(K * (K - 1))) + edge_b2_ref[...]
    node_agg = jnp.sum(v.reshape(G_CHUNK, K, NODE_DIM), axis=1) / float(K)

    g1 = jnp.maximum(
        jnp.dot(edge_agg, gW1e_ref[...], preferred_element_type=f32)
        + jnp.dot(node_agg, gW1n_ref[...], preferred_element_type=f32)
        + glob_b1_ref[...], 0.0)
    out_ref[...] = (jnp.dot(g1, glob_W2_ref[...], preferred_element_type=f32)
                    + glob_b2_ref[...])


@jax.jit
def kernel(theta, enc_W1, enc_b1, enc_W2, enc_b2, edge_W1, edge_b1, edge_W2,
           edge_b2, node_W1, node_b1, node_W2, node_b2, glob_W1, glob_b1,
           glob_W2, glob_b2):
    n_chunks = B // G_CHUNK

    def row2d(bias):
        return bias.reshape(1, -1)

    rep = lambda shape: pl.BlockSpec(shape, lambda i: (0, 0))

    grid_spec = pl.GridSpec(
        grid=(n_chunks,),
        in_specs=[
            pl.BlockSpec((ROWS, INPUT_DIM), lambda i: (i, 0)),
            rep((INPUT_DIM, LATENT_DIM)), rep((1, LATENT_DIM)),
            rep((LATENT_DIM, NODE_DIM)), rep((1, NODE_DIM)),
            rep((NODE_DIM, LATENT_DIM)), rep((NODE_DIM, LATENT_DIM)),
            rep((1, LATENT_DIM)),
            rep((LATENT_DIM, EDGE_DIM)), rep((1, EDGE_DIM)),
            rep((EDGE_DIM, LATENT_DIM)), rep((NODE_DIM, LATENT_DIM)),
            rep((1, LATENT_DIM)),
            rep((LATENT_DIM, NODE_DIM)), rep((1, NODE_DIM)),
            rep((EDGE_DIM, LATENT_DIM)), rep((NODE_DIM, LATENT_DIM)),
            rep((1, LATENT_DIM)),
            rep((LATENT_DIM, N_ACTIONS)), rep((1, N_ACTIONS)),
        ],
        out_specs=pl.BlockSpec((G_CHUNK, N_ACTIONS), lambda i: (i, 0)),
    )

    return pl.pallas_call(
        _gn_kernel,
        grid_spec=grid_spec,
        out_shape=jax.ShapeDtypeStruct((B, N_ACTIONS), jnp.float32),
        compiler_params=pltpu.CompilerParams(
            dimension_semantics=("arbitrary",)),
    )(
        theta,
        enc_W1, row2d(enc_b1), enc_W2, row2d(enc_b2),
        edge_W1[:NODE_DIM], edge_W1[NODE_DIM:], row2d(edge_b1),
        edge_W2, row2d(edge_b2),
        node_W1[:EDGE_DIM], node_W1[EDGE_DIM:], row2d(node_b1),
        node_W2, row2d(node_b2),
        glob_W1[:EDGE_DIM], glob_W1[EDGE_DIM:], row2d(glob_b1),
        glob_W2, row2d(glob_b2),
    )


# speedup vs baseline: 80.9221x; 1.2737x over previous
import functools

import jax
import jax.numpy as jnp
from jax.experimental import pallas as pl
from jax.experimental.pallas import tpu as pltpu

B = 256
K = 32
INPUT_DIM = 128
LATENT_DIM = 256
NODE_DIM = 128
EDGE_DIM = 128
N_ACTIONS = 32
N = B * K

G_CHUNK = 64
ROWS = G_CHUNK * K


def _gn_kernel(theta_ref,
               enc_W1_ref, enc_b1_ref, enc_W2_ref, enc_b2_ref,
               eW1r_ref, eW1s_ref, edge_b1_ref, edge_W2_ref, edge_b2_ref,
               nW1e_ref, nW1h_ref, node_b1_ref, node_W2_ref, node_b2_ref,
               gW1e_ref, gW1n_ref, glob_b1_ref, glob_W2_ref, glob_b2_ref,
               out_ref):
    f32 = jnp.float32

    h1 = jnp.maximum(
        jnp.dot(theta_ref[...], enc_W1_ref[...], preferred_element_type=f32)
        + enc_b1_ref[...], 0.0)
    h = jnp.dot(h1, enc_W2_ref[...], preferred_element_type=f32) + enc_b2_ref[...]

    a = jnp.dot(h, eW1r_ref[...], preferred_element_type=f32) + edge_b1_ref[...]
    b = jnp.dot(h, eW1s_ref[...], preferred_element_type=f32)

    a3 = a.reshape(G_CHUNK, K, LATENT_DIM)
    b3 = b.reshape(G_CHUNK, K, LATENT_DIM)

    a3h = a3.astype(jnp.bfloat16)
    b3h = b3.astype(jnp.bfloat16)
    acc0 = jnp.zeros(a3h.shape, jnp.bfloat16)
    acc1 = jnp.zeros(a3h.shape, jnp.bfloat16)
    for s in range(K // 2):
        acc0 = acc0 + jnp.maximum(a3h + b3h[:, s:s + 1, :], 0)
        acc1 = acc1 + jnp.maximum(a3h + b3h[:, s + K // 2:s + K // 2 + 1, :], 0)
    acc = (acc0.astype(f32) + acc1.astype(f32)) - jnp.maximum(a3 + b3, 0.0)
    R = acc.reshape(ROWS, LATENT_DIM)

    recv_mean = (jnp.dot(R, edge_W2_ref[...], preferred_element_type=f32)
                 / float(K - 1)) + edge_b2_ref[...]

    v1 = jnp.maximum(
        jnp.dot(recv_mean, nW1e_ref[...], preferred_element_type=f32)
        + jnp.dot(h, nW1h_ref[...], preferred_element_type=f32)
        + node_b1_ref[...], 0.0)
    v = jnp.dot(v1, node_W2_ref[...], preferred_element_type=f32) + node_b2_ref[...]

    Rsum = jnp.sum(R.reshape(G_CHUNK, K, LATENT_DIM), axis=1)
    edge_agg = (jnp.dot(Rsum, edge_W2_ref[...], preferred_element_type=f32)
                / float(K * (K - 1))) + edge_b2_ref[...]
    node_agg = jnp.sum(v.reshape(G_CHUNK, K, NODE_DIM), axis=1) / float(K)

    g1 = jnp.maximum(
        jnp.dot(edge_agg, gW1e_ref[...], preferred_element_type=f32)
        + jnp.dot(node_agg, gW1n_ref[...], preferred_element_type=f32)
        + glob_b1_ref[...], 0.0)
    out_ref[...] = (jnp.dot(g1, glob_W2_ref[...], preferred_element_type=f32)
                    + glob_b2_ref[...])


@jax.jit
def kernel(theta, enc_W1, enc_b1, enc_W2, enc_b2, edge_W1, edge_b1, edge_W2,
           edge_b2, node_W1, node_b1, node_W2, node_b2, glob_W1, glob_b1,
           glob_W2, glob_b2):
    n_chunks = B // G_CHUNK

    def row2d(bias):
        return bias.reshape(1, -1)

    rep = lambda shape: pl.BlockSpec(shape, lambda i: (0, 0))

    grid_spec = pl.GridSpec(
        grid=(n_chunks,),
        in_specs=[
            pl.BlockSpec((ROWS, INPUT_DIM), lambda i: (i, 0)),
            rep((INPUT_DIM, LATENT_DIM)), rep((1, LATENT_DIM)),
            rep((LATENT_DIM, NODE_DIM)), rep((1, NODE_DIM)),
            rep((NODE_DIM, LATENT_DIM)), rep((NODE_DIM, LATENT_DIM)),
            rep((1, LATENT_DIM)),
            rep((LATENT_DIM, EDGE_DIM)), rep((1, EDGE_DIM)),
            rep((EDGE_DIM, LATENT_DIM)), rep((NODE_DIM, LATENT_DIM)),
            rep((1, LATENT_DIM)),
            rep((LATENT_DIM, NODE_DIM)), rep((1, NODE_DIM)),
            rep((EDGE_DIM, LATENT_DIM)), rep((NODE_DIM, LATENT_DIM)),
            rep((1, LATENT_DIM)),
            rep((LATENT_DIM, N_ACTIONS)), rep((1, N_ACTIONS)),
        ],
        out_specs=pl.BlockSpec((G_CHUNK, N_ACTIONS), lambda i: (i, 0)),
    )

    return pl.pallas_call(
        _gn_kernel,
        grid_spec=grid_spec,
        out_shape=jax.ShapeDtypeStruct((B, N_ACTIONS), jnp.float32),
        compiler_params=pltpu.CompilerParams(
            dimension_semantics=("arbitrary",)),
    )(
        theta,
        enc_W1, row2d(enc_b1), enc_W2, row2d(enc_b2),
        edge_W1[:NODE_DIM], edge_W1[NODE_DIM:], row2d(edge_b1),
        edge_W2, row2d(edge_b2),
        node_W1[:EDGE_DIM], node_W1[EDGE_DIM:], row2d(node_b1),
        node_W2, row2d(node_b2),
        glob_W1[:EDGE_DIM], glob_W1[EDGE_DIM:], row2d(glob_b1),
        glob_W2, row2d(glob_b2),
    )


# speedup vs baseline: 83.3143x; 1.0296x over previous
import functools

import jax
import jax.numpy as jnp
from jax.experimental import pallas as pl
from jax.experimental.pallas import tpu as pltpu

B = 256
K = 32
INPUT_DIM = 128
LATENT_DIM = 256
NODE_DIM = 128
EDGE_DIM = 128
N_ACTIONS = 32
N = B * K

G_CHUNK = 128
ROWS = G_CHUNK * K


def _gn_kernel(theta_ref,
               enc_W1_ref, enc_b1_ref, enc_W2_ref, enc_b2_ref,
               eW1r_ref, eW1s_ref, edge_b1_ref, edge_W2_ref, edge_b2_ref,
               nW1e_ref, nW1h_ref, node_b1_ref, node_W2_ref, node_b2_ref,
               gW1e_ref, gW1n_ref, glob_b1_ref, glob_W2_ref, glob_b2_ref,
               out_ref):
    f32 = jnp.float32

    h1 = jnp.maximum(
        jnp.dot(theta_ref[...], enc_W1_ref[...], preferred_element_type=f32)
        + enc_b1_ref[...], 0.0)
    h = jnp.dot(h1, enc_W2_ref[...], preferred_element_type=f32) + enc_b2_ref[...]

    a = jnp.dot(h, eW1r_ref[...], preferred_element_type=f32) + edge_b1_ref[...]
    b = jnp.dot(h, eW1s_ref[...], preferred_element_type=f32)

    a3 = a.reshape(G_CHUNK, K, LATENT_DIM)
    b3 = b.reshape(G_CHUNK, K, LATENT_DIM)

    a3h = a3.astype(jnp.bfloat16)
    b3h = b3.astype(jnp.bfloat16)
    acc0 = jnp.zeros(a3h.shape, jnp.bfloat16)
    acc1 = jnp.zeros(a3h.shape, jnp.bfloat16)
    for s in range(K // 2):
        acc0 = acc0 + jnp.maximum(a3h + b3h[:, s:s + 1, :], 0)
        acc1 = acc1 + jnp.maximum(a3h + b3h[:, s + K // 2:s + K // 2 + 1, :], 0)
    acc = (acc0.astype(f32) + acc1.astype(f32)) - jnp.maximum(a3 + b3, 0.0)
    R = acc.reshape(ROWS, LATENT_DIM)

    recv_mean = (jnp.dot(R, edge_W2_ref[...], preferred_element_type=f32)
                 / float(K - 1)) + edge_b2_ref[...]

    v1 = jnp.maximum(
        jnp.dot(recv_mean, nW1e_ref[...], preferred_element_type=f32)
        + jnp.dot(h, nW1h_ref[...], preferred_element_type=f32)
        + node_b1_ref[...], 0.0)
    v = jnp.dot(v1, node_W2_ref[...], preferred_element_type=f32) + node_b2_ref[...]

    Rsum = jnp.sum(R.reshape(G_CHUNK, K, LATENT_DIM), axis=1)
    edge_agg = (jnp.dot(Rsum, edge_W2_ref[...], preferred_element_type=f32)
                / float(K * (K - 1))) + edge_b2_ref[...]
    node_agg = jnp.sum(v.reshape(G_CHUNK, K, NODE_DIM), axis=1) / float(K)

    g1 = jnp.maximum(
        jnp.dot(edge_agg, gW1e_ref[...], preferred_element_type=f32)
        + jnp.dot(node_agg, gW1n_ref[...], preferred_element_type=f32)
        + glob_b1_ref[...], 0.0)
    out_ref[...] = (jnp.dot(g1, glob_W2_ref[...], preferred_element_type=f32)
                    + glob_b2_ref[...])


@jax.jit
def kernel(theta, enc_W1, enc_b1, enc_W2, enc_b2, edge_W1, edge_b1, edge_W2,
           edge_b2, node_W1, node_b1, node_W2, node_b2, glob_W1, glob_b1,
           glob_W2, glob_b2):
    n_chunks = B // G_CHUNK

    def row2d(bias):
        return bias.reshape(1, -1)

    rep = lambda shape: pl.BlockSpec(shape, lambda i: (0, 0))

    grid_spec = pl.GridSpec(
        grid=(n_chunks,),
        in_specs=[
            pl.BlockSpec((ROWS, INPUT_DIM), lambda i: (i, 0)),
            rep((INPUT_DIM, LATENT_DIM)), rep((1, LATENT_DIM)),
            rep((LATENT_DIM, NODE_DIM)), rep((1, NODE_DIM)),
            rep((NODE_DIM, LATENT_DIM)), rep((NODE_DIM, LATENT_DIM)),
            rep((1, LATENT_DIM)),
            rep((LATENT_DIM, EDGE_DIM)), rep((1, EDGE_DIM)),
            rep((EDGE_DIM, LATENT_DIM)), rep((NODE_DIM, LATENT_DIM)),
            rep((1, LATENT_DIM)),
            rep((LATENT_DIM, NODE_DIM)), rep((1, NODE_DIM)),
            rep((EDGE_DIM, LATENT_DIM)), rep((NODE_DIM, LATENT_DIM)),
            rep((1, LATENT_DIM)),
            rep((LATENT_DIM, N_ACTIONS)), rep((1, N_ACTIONS)),
        ],
        out_specs=pl.BlockSpec((G_CHUNK, N_ACTIONS), lambda i: (i, 0)),
    )

    return pl.pallas_call(
        _gn_kernel,
        grid_spec=grid_spec,
        out_shape=jax.ShapeDtypeStruct((B, N_ACTIONS), jnp.float32),
        compiler_params=pltpu.CompilerParams(
            dimension_semantics=("arbitrary",)),
    )(
        theta,
        enc_W1, row2d(enc_b1), enc_W2, row2d(enc_b2),
        edge_W1[:NODE_DIM], edge_W1[NODE_DIM:], row2d(edge_b1),
        edge_W2, row2d(edge_b2),
        node_W1[:EDGE_DIM], node_W1[EDGE_DIM:], row2d(node_b1),
        node_W2, row2d(node_b2),
        glob_W1[:EDGE_DIM], glob_W1[EDGE_DIM:], row2d(glob_b1),
        glob_W2, row2d(glob_b2),
    )


# speedup vs baseline: 91.8086x; 1.1020x over previous
import jax
import jax.numpy as jnp
from jax.experimental import pallas as pl
from jax.experimental.pallas import tpu as pltpu

B = 256
K = 32
INPUT_DIM = 128
LATENT_DIM = 256
NODE_DIM = 128
EDGE_DIM = 128
N_ACTIONS = 32
N = B * K

G_CHUNK = 128
ROWS = G_CHUNK * K


def _gn_kernel(theta_ref,
               enc_W1_ref, enc_b1_ref, enc_W2_ref, enc_b2_ref,
               edge_W1_ref, edge_b1_ref, edge_W2_ref, edge_b2_ref,
               node_W1_ref, node_b1_ref, node_W2_ref, node_b2_ref,
               glob_W1_ref, glob_b1_ref, glob_W2_ref, glob_b2_ref,
               out_ref):
    f32 = jnp.float32

    h1 = jnp.maximum(
        jnp.dot(theta_ref[...], enc_W1_ref[...], preferred_element_type=f32)
        + enc_b1_ref[...], 0.0)
    h = jnp.dot(h1, enc_W2_ref[...], preferred_element_type=f32) + enc_b2_ref[...]

    a = (jnp.dot(h, edge_W1_ref[:NODE_DIM, :], preferred_element_type=f32)
         + edge_b1_ref[...])
    b = jnp.dot(h, edge_W1_ref[NODE_DIM:, :], preferred_element_type=f32)

    a3 = a.reshape(G_CHUNK, K, LATENT_DIM)
    b3 = b.reshape(G_CHUNK, K, LATENT_DIM)

    a3h = a3.astype(jnp.bfloat16)
    b3h = b3.astype(jnp.bfloat16)
    acc0 = jnp.zeros(a3h.shape, jnp.bfloat16)
    acc1 = jnp.zeros(a3h.shape, jnp.bfloat16)
    for s in range(K // 2):
        acc0 = acc0 + jnp.maximum(a3h + b3h[:, s:s + 1, :], 0)
        acc1 = acc1 + jnp.maximum(a3h + b3h[:, s + K // 2:s + K // 2 + 1, :], 0)
    acc = (acc0.astype(f32) + acc1.astype(f32)) - jnp.maximum(a3 + b3, 0.0)
    R = acc.reshape(ROWS, LATENT_DIM)

    recv_mean = (jnp.dot(R, edge_W2_ref[...], preferred_element_type=f32)
                 / float(K - 1)) + edge_b2_ref[...]

    v1 = jnp.maximum(
        jnp.dot(recv_mean, node_W1_ref[:EDGE_DIM, :], preferred_element_type=f32)
        + jnp.dot(h, node_W1_ref[EDGE_DIM:, :], preferred_element_type=f32)
        + node_b1_ref[...], 0.0)
    v = jnp.dot(v1, node_W2_ref[...], preferred_element_type=f32) + node_b2_ref[...]

    Rsum = jnp.sum(R.reshape(G_CHUNK, K, LATENT_DIM), axis=1)
    edge_agg = (jnp.dot(Rsum, edge_W2_ref[...], preferred_element_type=f32)
                / float(K * (K - 1))) + edge_b2_ref[...]
    node_agg = jnp.sum(v.reshape(G_CHUNK, K, NODE_DIM), axis=1) / float(K)

    g1 = jnp.maximum(
        jnp.dot(edge_agg, glob_W1_ref[:EDGE_DIM, :], preferred_element_type=f32)
        + jnp.dot(node_agg, glob_W1_ref[EDGE_DIM:, :], preferred_element_type=f32)
        + glob_b1_ref[...], 0.0)
    out_ref[...] = (jnp.dot(g1, glob_W2_ref[...], preferred_element_type=f32)
                    + glob_b2_ref[...])


@jax.jit
def kernel(theta, enc_W1, enc_b1, enc_W2, enc_b2, edge_W1, edge_b1, edge_W2,
           edge_b2, node_W1, node_b1, node_W2, node_b2, glob_W1, glob_b1,
           glob_W2, glob_b2):
    n_chunks = B // G_CHUNK

    def row2d(bias):
        return bias.reshape(1, -1)

    rep = lambda shape: pl.BlockSpec(shape, lambda i: (0, 0))

    grid_spec = pl.GridSpec(
        grid=(n_chunks,),
        in_specs=[
            pl.BlockSpec((ROWS, INPUT_DIM), lambda i: (i, 0)),
            rep((INPUT_DIM, LATENT_DIM)), rep((1, LATENT_DIM)),
            rep((LATENT_DIM, NODE_DIM)), rep((1, NODE_DIM)),
            rep((2 * NODE_DIM, LATENT_DIM)), rep((1, LATENT_DIM)),
            rep((LATENT_DIM, EDGE_DIM)), rep((1, EDGE_DIM)),
            rep((NODE_DIM + EDGE_DIM, LATENT_DIM)), rep((1, LATENT_DIM)),
            rep((LATENT_DIM, NODE_DIM)), rep((1, NODE_DIM)),
            rep((NODE_DIM + EDGE_DIM, LATENT_DIM)), rep((1, LATENT_DIM)),
            rep((LATENT_DIM, N_ACTIONS)), rep((1, N_ACTIONS)),
        ],
        out_specs=pl.BlockSpec((G_CHUNK, N_ACTIONS), lambda i: (i, 0)),
    )

    return pl.pallas_call(
        _gn_kernel,
        grid_spec=grid_spec,
        out_shape=jax.ShapeDtypeStruct((B, N_ACTIONS), jnp.float32),
        compiler_params=pltpu.CompilerParams(
            dimension_semantics=("arbitrary",)),
    )(
        theta,
        enc_W1, row2d(enc_b1), enc_W2, row2d(enc_b2),
        edge_W1, row2d(edge_b1), edge_W2, row2d(edge_b2),
        node_W1, row2d(node_b1), node_W2, row2d(node_b2),
        glob_W1, row2d(glob_b1), glob_W2, row2d(glob_b2),
    )


# speedup vs baseline: 99.8939x; 1.0881x over previous
import jax
import jax.numpy as jnp
from jax.experimental import pallas as pl
from jax.experimental.pallas import tpu as pltpu

B = 256
K = 32
INPUT_DIM = 128
LATENT_DIM = 256
NODE_DIM = 128
EDGE_DIM = 128
N_ACTIONS = 32
N = B * K

G_CHUNK = 128
ROWS = G_CHUNK * K


def _gn_kernel(theta_ref,
               enc_W1_ref, enc_b1_ref, enc_W2_ref, enc_b2_ref,
               edge_W1_ref, edge_b1_ref, edge_W2_ref, edge_b2_ref,
               node_W1_ref, node_b1_ref, node_W2_ref, node_b2_ref,
               glob_W1_ref, glob_b1_ref, glob_W2_ref, glob_b2_ref,
               out_ref):
    f32 = jnp.float32
    bf16 = jnp.bfloat16

    def bdot(x, w):
        return jnp.dot(x.astype(bf16), w.astype(bf16),
                       preferred_element_type=f32)

    h1 = jnp.maximum(bdot(theta_ref[...], enc_W1_ref[...]) + enc_b1_ref[...],
                     0.0)
    h = bdot(h1, enc_W2_ref[...]) + enc_b2_ref[...]

    hh = h.astype(bf16)
    a = bdot(hh, edge_W1_ref[:NODE_DIM, :]) + edge_b1_ref[...]
    b = bdot(hh, edge_W1_ref[NODE_DIM:, :])

    a3h = a.astype(bf16).reshape(G_CHUNK, K, LATENT_DIM)
    b3h = b.astype(bf16).reshape(G_CHUNK, K, LATENT_DIM)
    acc0 = -jnp.maximum(a3h + b3h, 0)
    acc1 = jnp.zeros(a3h.shape, bf16)
    for s in range(K // 2):
        acc0 = acc0 + jnp.maximum(a3h + b3h[:, s:s + 1, :], 0)
        acc1 = acc1 + jnp.maximum(a3h + b3h[:, s + K // 2:s + K // 2 + 1, :], 0)
    R = (acc0 + acc1).reshape(ROWS, LATENT_DIM)

    recv_mean = bdot(R, edge_W2_ref[...]) / float(K - 1) + edge_b2_ref[...]

    v1 = jnp.maximum(
        bdot(recv_mean, node_W1_ref[:EDGE_DIM, :])
        + bdot(hh, node_W1_ref[EDGE_DIM:, :])
        + node_b1_ref[...], 0.0)
    v = bdot(v1, node_W2_ref[...]) + node_b2_ref[...]

    edge_agg = jnp.sum(recv_mean.reshape(G_CHUNK, K, EDGE_DIM), axis=1) / float(K)
    node_agg = jnp.sum(v.reshape(G_CHUNK, K, NODE_DIM), axis=1) / float(K)

    g1 = jnp.maximum(
        bdot(edge_agg, glob_W1_ref[:EDGE_DIM, :])
        + bdot(node_agg, glob_W1_ref[EDGE_DIM:, :])
        + glob_b1_ref[...], 0.0)
    out_ref[...] = bdot(g1, glob_W2_ref[...]) + glob_b2_ref[...]


@jax.jit
def kernel(theta, enc_W1, enc_b1, enc_W2, enc_b2, edge_W1, edge_b1, edge_W2,
           edge_b2, node_W1, node_b1, node_W2, node_b2, glob_W1, glob_b1,
           glob_W2, glob_b2):
    n_chunks = B // G_CHUNK

    rep = lambda shape: pl.BlockSpec(shape, lambda i: (0,) * len(shape))

    grid_spec = pl.GridSpec(
        grid=(n_chunks,),
        in_specs=[
            pl.BlockSpec((ROWS, INPUT_DIM), lambda i: (i, 0)),
            rep((INPUT_DIM, LATENT_DIM)), rep((LATENT_DIM,)),
            rep((LATENT_DIM, NODE_DIM)), rep((NODE_DIM,)),
            rep((2 * NODE_DIM, LATENT_DIM)), rep((LATENT_DIM,)),
            rep((LATENT_DIM, EDGE_DIM)), rep((EDGE_DIM,)),
            rep((NODE_DIM + EDGE_DIM, LATENT_DIM)), rep((LATENT_DIM,)),
            rep((LATENT_DIM, NODE_DIM)), rep((NODE_DIM,)),
            rep((NODE_DIM + EDGE_DIM, LATENT_DIM)), rep((LATENT_DIM,)),
            rep((LATENT_DIM, N_ACTIONS)), rep((N_ACTIONS,)),
        ],
        out_specs=pl.BlockSpec((G_CHUNK, N_ACTIONS), lambda i: (i, 0)),
    )

    return pl.pallas_call(
        _gn_kernel,
        grid_spec=grid_spec,
        out_shape=jax.ShapeDtypeStruct((B, N_ACTIONS), jnp.float32),
        compiler_params=pltpu.CompilerParams(
            dimension_semantics=("arbitrary",)),
    )(
        theta,
        enc_W1, enc_b1, enc_W2, enc_b2,
        edge_W1, edge_b1, edge_W2, edge_b2,
        node_W1, node_b1, node_W2, node_b2,
        glob_W1, glob_b1, glob_W2, glob_b2,
    )


# speedup vs baseline: 101.7575x; 1.0187x over previous
import jax
import jax.numpy as jnp
from jax.experimental import pallas as pl
from jax.experimental.pallas import tpu as pltpu

B = 256
K = 32
INPUT_DIM = 128
LATENT_DIM = 256
NODE_DIM = 128
EDGE_DIM = 128
N_ACTIONS = 32
N = B * K

G_CHUNK = 128
ROWS = G_CHUNK * K


def _gn_kernel(theta_ref,
               enc_W1_ref, enc_b1_ref, enc_W2_ref, enc_b2_ref,
               edge_W1_ref, edge_b1_ref, edge_W2_ref, edge_b2_ref,
               node_W1_ref, node_b1_ref, node_W2_ref, node_b2_ref,
               glob_W1_ref, glob_b1_ref, glob_W2_ref, glob_b2_ref,
               out_ref):
    f32 = jnp.float32
    bf16 = jnp.bfloat16

    def bdot16(x, w):
        return jnp.dot(x, w.astype(bf16),
                       preferred_element_type=f32).astype(bf16)

    def bias16(bias_ref):
        return bias_ref[...].astype(bf16)

    th = theta_ref[...].astype(bf16)
    h1 = jnp.maximum(bdot16(th, enc_W1_ref[...]) + bias16(enc_b1_ref), 0)
    h = bdot16(h1, enc_W2_ref[...]) + bias16(enc_b2_ref)

    a = bdot16(h, edge_W1_ref[:NODE_DIM, :]) + bias16(edge_b1_ref)
    b = bdot16(h, edge_W1_ref[NODE_DIM:, :])

    a3h = a.reshape(G_CHUNK, K, LATENT_DIM)
    b3h = b.reshape(G_CHUNK, K, LATENT_DIM)
    acc0 = -jnp.maximum(a3h + b3h, 0)
    acc1 = jnp.zeros(a3h.shape, bf16)
    for s in range(K // 2):
        acc0 = acc0 + jnp.maximum(a3h + b3h[:, s:s + 1, :], 0)
        acc1 = acc1 + jnp.maximum(a3h + b3h[:, s + K // 2:s + K // 2 + 1, :], 0)
    R = (acc0 + acc1).reshape(ROWS, LATENT_DIM)

    eW2s = (edge_W2_ref[...] * (1.0 / float(K - 1))).astype(bf16)
    recv_mean = (jnp.dot(R, eW2s, preferred_element_type=f32).astype(bf16)
                 + bias16(edge_b2_ref))

    v1 = jnp.maximum(
        bdot16(recv_mean, node_W1_ref[:EDGE_DIM, :])
        + bdot16(h, node_W1_ref[EDGE_DIM:, :])
        + bias16(node_b1_ref), 0)

    edge_agg = jnp.sum(recv_mean.reshape(G_CHUNK, K, EDGE_DIM), axis=1) * bf16(1.0 / K)
    v1m = jnp.sum(v1.reshape(G_CHUNK, K, LATENT_DIM), axis=1) * bf16(1.0 / K)
    node_agg = bdot16(v1m, node_W2_ref[...]) + bias16(node_b2_ref)

    g1 = jnp.maximum(
        bdot16(edge_agg, glob_W1_ref[:EDGE_DIM, :])
        + bdot16(node_agg, glob_W1_ref[EDGE_DIM:, :])
        + bias16(glob_b1_ref), 0)
    out_ref[...] = (jnp.dot(g1, glob_W2_ref[...].astype(bf16),
                            preferred_element_type=f32)
                    + glob_b2_ref[...])


@jax.jit
def kernel(theta, enc_W1, enc_b1, enc_W2, enc_b2, edge_W1, edge_b1, edge_W2,
           edge_b2, node_W1, node_b1, node_W2, node_b2, glob_W1, glob_b1,
           glob_W2, glob_b2):
    n_chunks = B // G_CHUNK

    rep = lambda shape: pl.BlockSpec(shape, lambda i: (0,) * len(shape))

    grid_spec = pl.GridSpec(
        grid=(n_chunks,),
        in_specs=[
            pl.BlockSpec((ROWS, INPUT_DIM), lambda i: (i, 0)),
            rep((INPUT_DIM, LATENT_DIM)), rep((LATENT_DIM,)),
            rep((LATENT_DIM, NODE_DIM)), rep((NODE_DIM,)),
            rep((2 * NODE_DIM, LATENT_DIM)), rep((LATENT_DIM,)),
            rep((LATENT_DIM, EDGE_DIM)), rep((EDGE_DIM,)),
            rep((NODE_DIM + EDGE_DIM, LATENT_DIM)), rep((LATENT_DIM,)),
            rep((LATENT_DIM, NODE_DIM)), rep((NODE_DIM,)),
            rep((NODE_DIM + EDGE_DIM, LATENT_DIM)), rep((LATENT_DIM,)),
            rep((LATENT_DIM, N_ACTIONS)), rep((N_ACTIONS,)),
        ],
        out_specs=pl.BlockSpec((G_CHUNK, N_ACTIONS), lambda i: (i, 0)),
    )

    return pl.pallas_call(
        _gn_kernel,
        grid_spec=grid_spec,
        out_shape=jax.ShapeDtypeStruct((B, N_ACTIONS), jnp.float32),
        compiler_params=pltpu.CompilerParams(
            dimension_semantics=("arbitrary",)),
    )(
        theta,
        enc_W1, enc_b1, enc_W2, enc_b2,
        edge_W1, edge_b1, edge_W2, edge_b2,
        node_W1, node_b1, node_W2, node_b2,
        glob_W1, glob_b1, glob_W2, glob_b2,
    )
